# baseline (speedup 1.0000x reference)
import functools

import jax
import jax.numpy as jnp
from jax import lax
from jax.experimental import pallas as pl
from jax.experimental.pallas import tpu as pltpu
from jax.experimental.pallas import tpu_sc as plsc

N = 10000
E = 320000
D = 128
STEPS = 8

CHUNK = 128
NC, NS = 2, 16
NW = NC * NS
RPW = 320
NPAD = RPW * NW
NBOUND = 272

ROWB = 2000
NROWB = N // ROWB


def _sc_segsum_body(hw, srcs, dsts, bounds, zrows, out,
                    src_v, dst_v, rows_v, bnd_v, acc, sem):
    c = lax.axis_index("c")
    s = lax.axis_index("s")
    w = s * NC + c
    pltpu.sync_copy(bounds, bnd_v)
    pltpu.sync_copy(zrows, acc)
    e_lo = bnd_v[pl.ds(pl.multiple_of(w * 8, 8), 16)][0]
    e_hi = bnd_v[pl.ds(pl.multiple_of((w + 1) * 8, 8), 16)][0]
    row_base = pl.multiple_of(w * RPW, 8)

    def chunk_body(base):
        base = pl.multiple_of(base, CHUNK)
        pltpu.sync_copy(srcs.at[pl.ds(base, CHUNK)], src_v)
        pltpu.sync_copy(dsts.at[pl.ds(base, CHUNK)], dst_v)
        pltpu.async_copy(hw.at[src_v], rows_v, sem).wait()

        def edge_work(g, j):
            r = dst_v[pl.ds(pl.multiple_of(g * 16, 16), 16)][j] - row_base
            for k in range(D // 16):
                plsc.addupdate(acc.at[r, pl.ds(k * 16, 16)],
                               rows_v[g * 16 + j, pl.ds(k * 16, 16)])

        @pl.when(jnp.logical_and(base >= e_lo, base + CHUNK <= e_hi))
        def _interior():
            def group_body(g, carry):
                for j in range(16):
                    edge_work(g, j)
                return carry

            lax.fori_loop(0, CHUNK // 16, group_body, 0)

        @pl.when(jnp.logical_or(base < e_lo, base + CHUNK > e_hi))
        def _boundary():
            def group_body(g, carry):
                for j in range(16):
                    e = base + g * 16 + j

                    @pl.when(jnp.logical_and(e >= e_lo, e < e_hi))
                    def _():
                        edge_work(g, j)

                return carry

            lax.fori_loop(0, CHUNK // 16, group_body, 0)

    first = (e_lo // CHUNK) * CHUNK

    def outer_body(i, carry):
        base = first + i * CHUNK

        @pl.when(base < e_hi)
        def _():
            chunk_body(base)

        return carry

    lax.fori_loop(0, E // CHUNK, outer_body, 0)
    pltpu.sync_copy(acc, out.at[pl.ds(row_base, RPW), :])


@functools.cache
def _get_sc_segsum():
    return pl.kernel(
        _sc_segsum_body,
        out_type=jax.ShapeDtypeStruct((NPAD, D), jnp.float32),
        mesh=plsc.VectorSubcoreMesh(core_axis_name="c", subcore_axis_name="s",
                                    num_cores=NC, num_subcores=NS),
        scratch_types=[
            pltpu.VMEM((CHUNK,), jnp.int32),
            pltpu.VMEM((CHUNK,), jnp.int32),
            pltpu.VMEM((CHUNK, D), jnp.float32),
            pltpu.VMEM((NBOUND,), jnp.int32),
            pltpu.VMEM((RPW, D), jnp.float32),
            pltpu.SemaphoreType.DMA,
        ],
    )


def _tca_body(h_ref, lwT_ref, lb_ref, whhT_ref, bhh_ref, hw_ref, gh_ref):
    h = h_ref[...]
    hw_ref[...] = jnp.dot(h, lwT_ref[...],
                          preferred_element_type=jnp.float32) + lb_ref[...]
    gh_ref[...] = jnp.dot(h, whhT_ref[...],
                          preferred_element_type=jnp.float32) + bhh_ref[...]


_tca = pl.pallas_call(
    _tca_body,
    grid=(NROWB,),
    in_specs=[
        pl.BlockSpec((ROWB, D), lambda i: (i, 0)),
        pl.BlockSpec((D, D), lambda i: (0, 0)),
        pl.BlockSpec((1, D), lambda i: (0, 0)),
        pl.BlockSpec((D, 3 * D), lambda i: (0, 0)),
        pl.BlockSpec((1, 3 * D), lambda i: (0, 0)),
    ],
    out_specs=[
        pl.BlockSpec((ROWB, D), lambda i: (i, 0)),
        pl.BlockSpec((ROWB, 3 * D), lambda i: (i, 0)),
    ],
    out_shape=[
        jax.ShapeDtypeStruct((N, D), jnp.float32),
        jax.ShapeDtypeStruct((N, 3 * D), jnp.float32),
    ],
)


def _tcb_body(a_ref, gh_ref, h_ref, wihT_ref, bih_ref, o_ref, *, relu_out):
    gi = jnp.dot(a_ref[...], wihT_ref[...],
                 preferred_element_type=jnp.float32) + bih_ref[...]
    gh = gh_ref[...]
    r = jax.nn.sigmoid(gi[:, :D] + gh[:, :D])
    z = jax.nn.sigmoid(gi[:, D:2 * D] + gh[:, D:2 * D])
    ng = jnp.tanh(gi[:, 2 * D:] + r * gh[:, 2 * D:])
    hn = (1.0 - z) * ng + z * h_ref[...]
    if relu_out:
        hn = jnp.maximum(hn, 0.0)
    o_ref[...] = hn


def _make_tcb(relu_out):
    return pl.pallas_call(
        functools.partial(_tcb_body, relu_out=relu_out),
        grid=(NROWB,),
        in_specs=[
            pl.BlockSpec((ROWB, D), lambda i: (i, 0)),
            pl.BlockSpec((ROWB, 3 * D), lambda i: (i, 0)),
            pl.BlockSpec((ROWB, D), lambda i: (i, 0)),
            pl.BlockSpec((D, 3 * D), lambda i: (0, 0)),
            pl.BlockSpec((1, 3 * D), lambda i: (0, 0)),
        ],
        out_specs=pl.BlockSpec((ROWB, D), lambda i: (i, 0)),
        out_shape=jax.ShapeDtypeStruct((N, D), jnp.float32),
    )


_tcb = _make_tcb(False)
_tcb_relu = _make_tcb(True)


def _mean_body(h_ref, o_ref):
    @pl.when(pl.program_id(0) == 0)
    def _():
        o_ref[...] = jnp.zeros_like(o_ref)

    o_ref[...] += jnp.sum(h_ref[...], axis=0, keepdims=True) * (1.0 / N)


_mean = pl.pallas_call(
    _mean_body,
    grid=(NROWB,),
    in_specs=[pl.BlockSpec((ROWB, D), lambda i: (i, 0))],
    out_specs=pl.BlockSpec((1, D), lambda i: (0, 0)),
    out_shape=jax.ShapeDtypeStruct((1, D), jnp.float32),
)


def kernel(in_feat, edge_index, lin_w1, lin_b1, w_ih1, w_hh1, b_ih1, b_hh1,
           lin_w2, lin_b2, w_ih2, w_hh2, b_ih2, b_hh2):
    src = edge_index[0]
    dst = edge_index[1]
    perm = jnp.argsort(dst, stable=True)
    srcs = src[perm].astype(jnp.int32)
    dsts = dst[perm].astype(jnp.int32)
    edges = jnp.arange(0, NPAD + 1, RPW, dtype=jnp.int32)
    bounds = jnp.searchsorted(dsts, edges[:NW + 1]).astype(jnp.int32)
    bounds = jnp.repeat(bounds, 8)
    bounds = jnp.pad(bounds, (0, NBOUND - 8 * (NW + 1)), constant_values=E)
    zrows = jnp.zeros((RPW, D), jnp.float32)

    h = in_feat
    params = (
        (lin_w1.T, lin_b1.reshape(1, D), w_ih1.T, w_hh1.T,
         b_ih1.reshape(1, 3 * D), b_hh1.reshape(1, 3 * D)),
        (lin_w2.T, lin_b2.reshape(1, D), w_ih2.T, w_hh2.T,
         b_ih2.reshape(1, 3 * D), b_hh2.reshape(1, 3 * D)),
    )
    for conv in (0, 1):
        lwT, lb, wihT, whhT, bih, bhh = params[conv]
        for step in range(STEPS):
            hw, gh = _tca(h, lwT, lb, whhT, bhh)
            a = _get_sc_segsum()(hw, srcs, dsts, bounds, zrows)
            step_fn = _tcb_relu if (conv == 0 and step == STEPS - 1) else _tcb
            h = step_fn(a, gh, h, wihT, bih)
    return _mean(h)


# speedup vs baseline: 1.1918x; 1.1918x over previous
import functools

import jax
import jax.numpy as jnp
from jax import lax
from jax.experimental import pallas as pl
from jax.experimental.pallas import tpu as pltpu
from jax.experimental.pallas import tpu_sc as plsc

N = 10000
E = 320000
D = 128
STEPS = 8

CHUNK = 128
NC, NS = 2, 16
NW = NC * NS
RPW = 320
NPAD = RPW * NW
NBOUND = 272

ROWB = 2000
NROWB = N // ROWB


def _sc_segsum_body(hw, srcs, dsts, bounds, zrows, out,
                    src_v, dst_v, rows_v, bnd_v, acc, sem):
    c = lax.axis_index("c")
    s = lax.axis_index("s")
    w = s * NC + c
    pltpu.sync_copy(bounds, bnd_v)
    pltpu.sync_copy(zrows, acc)
    e_lo = bnd_v[pl.ds(pl.multiple_of(w * 8, 8), 16)][0]
    e_hi = bnd_v[pl.ds(pl.multiple_of((w + 1) * 8, 8), 16)][0]
    row_base = pl.multiple_of(w * RPW, 8)

    def chunk_body(base):
        base = pl.multiple_of(base, CHUNK)
        pltpu.sync_copy(srcs.at[pl.ds(base, CHUNK)], src_v)
        pltpu.sync_copy(dsts.at[pl.ds(base, CHUNK)], dst_v)
        pltpu.async_copy(hw.at[src_v], rows_v, sem).wait()

        def group_body(g, carry):
            dvec = dst_v[pl.ds(pl.multiple_of(g * 16, 16), 16)]
            for j in range(16):
                e = base + g * 16 + j

                @pl.when(jnp.logical_and(e >= e_lo, e < e_hi))
                def _():
                    r = dvec[j] - row_base
                    for k in range(D // 16):
                        plsc.addupdate(acc.at[r, pl.ds(k * 16, 16)],
                                       rows_v[g * 16 + j, pl.ds(k * 16, 16)])

            return carry

        lax.fori_loop(0, CHUNK // 16, group_body, 0)

    first = (e_lo // CHUNK) * CHUNK

    def outer_body(i, carry):
        base = first + i * CHUNK

        @pl.when(base < e_hi)
        def _():
            chunk_body(base)

        return carry

    lax.fori_loop(0, E // CHUNK, outer_body, 0)
    pltpu.sync_copy(acc, out.at[pl.ds(row_base, RPW), :])


@functools.cache
def _get_sc_segsum():
    return pl.kernel(
        _sc_segsum_body,
        out_type=jax.ShapeDtypeStruct((NPAD, D), jnp.float32),
        mesh=plsc.VectorSubcoreMesh(core_axis_name="c", subcore_axis_name="s",
                                    num_cores=NC, num_subcores=NS),
        scratch_types=[
            pltpu.VMEM((CHUNK,), jnp.int32),
            pltpu.VMEM((CHUNK,), jnp.int32),
            pltpu.VMEM((CHUNK, D), jnp.float32),
            pltpu.VMEM((NBOUND,), jnp.int32),
            pltpu.VMEM((RPW, D), jnp.float32),
            pltpu.SemaphoreType.DMA,
        ],
    )


def _tca_body(h_ref, lwT_ref, lb_ref, whhT_ref, bhh_ref, hw_ref, gh_ref):
    h = h_ref[...]
    hw_ref[...] = jnp.dot(h, lwT_ref[...],
                          preferred_element_type=jnp.float32) + lb_ref[...]
    gh_ref[...] = jnp.dot(h, whhT_ref[...],
                          preferred_element_type=jnp.float32) + bhh_ref[...]


_tca = pl.pallas_call(
    _tca_body,
    grid=(NROWB,),
    in_specs=[
        pl.BlockSpec((ROWB, D), lambda i: (i, 0)),
        pl.BlockSpec((D, D), lambda i: (0, 0)),
        pl.BlockSpec((1, D), lambda i: (0, 0)),
        pl.BlockSpec((D, 3 * D), lambda i: (0, 0)),
        pl.BlockSpec((1, 3 * D), lambda i: (0, 0)),
    ],
    out_specs=[
        pl.BlockSpec((ROWB, D), lambda i: (i, 0)),
        pl.BlockSpec((ROWB, 3 * D), lambda i: (i, 0)),
    ],
    out_shape=[
        jax.ShapeDtypeStruct((N, D), jnp.float32),
        jax.ShapeDtypeStruct((N, 3 * D), jnp.float32),
    ],
)


def _tcb_body(a_ref, gh_ref, h_ref, wihT_ref, bih_ref, o_ref, *, relu_out):
    gi = jnp.dot(a_ref[...], wihT_ref[...],
                 preferred_element_type=jnp.float32) + bih_ref[...]
    gh = gh_ref[...]
    r = jax.nn.sigmoid(gi[:, :D] + gh[:, :D])
    z = jax.nn.sigmoid(gi[:, D:2 * D] + gh[:, D:2 * D])
    ng = jnp.tanh(gi[:, 2 * D:] + r * gh[:, 2 * D:])
    hn = (1.0 - z) * ng + z * h_ref[...]
    if relu_out:
        hn = jnp.maximum(hn, 0.0)
    o_ref[...] = hn


def _make_tcb(relu_out):
    return pl.pallas_call(
        functools.partial(_tcb_body, relu_out=relu_out),
        grid=(NROWB,),
        in_specs=[
            pl.BlockSpec((ROWB, D), lambda i: (i, 0)),
            pl.BlockSpec((ROWB, 3 * D), lambda i: (i, 0)),
            pl.BlockSpec((ROWB, D), lambda i: (i, 0)),
            pl.BlockSpec((D, 3 * D), lambda i: (0, 0)),
            pl.BlockSpec((1, 3 * D), lambda i: (0, 0)),
        ],
        out_specs=pl.BlockSpec((ROWB, D), lambda i: (i, 0)),
        out_shape=jax.ShapeDtypeStruct((N, D), jnp.float32),
    )


_tcb = _make_tcb(False)
_tcb_relu = _make_tcb(True)


def _mean_body(h_ref, o_ref):
    @pl.when(pl.program_id(0) == 0)
    def _():
        o_ref[...] = jnp.zeros_like(o_ref)

    o_ref[...] += jnp.sum(h_ref[...], axis=0, keepdims=True) * (1.0 / N)


_mean = pl.pallas_call(
    _mean_body,
    grid=(NROWB,),
    in_specs=[pl.BlockSpec((ROWB, D), lambda i: (i, 0))],
    out_specs=pl.BlockSpec((1, D), lambda i: (0, 0)),
    out_shape=jax.ShapeDtypeStruct((1, D), jnp.float32),
)


def kernel(in_feat, edge_index, lin_w1, lin_b1, w_ih1, w_hh1, b_ih1, b_hh1,
           lin_w2, lin_b2, w_ih2, w_hh2, b_ih2, b_hh2):
    src = edge_index[0]
    dst = edge_index[1]
    perm = jnp.argsort(dst, stable=True)
    srcs = src[perm].astype(jnp.int32)
    dsts = dst[perm].astype(jnp.int32)
    edges = jnp.arange(0, NPAD + 1, RPW, dtype=jnp.int32)
    bounds = jnp.searchsorted(dsts, edges[:NW + 1]).astype(jnp.int32)
    bounds = jnp.repeat(bounds, 8)
    bounds = jnp.pad(bounds, (0, NBOUND - 8 * (NW + 1)), constant_values=E)
    zrows = jnp.zeros((RPW, D), jnp.float32)

    h = in_feat
    params = (
        (lin_w1.T, lin_b1.reshape(1, D), w_ih1.T, w_hh1.T,
         b_ih1.reshape(1, 3 * D), b_hh1.reshape(1, 3 * D)),
        (lin_w2.T, lin_b2.reshape(1, D), w_ih2.T, w_hh2.T,
         b_ih2.reshape(1, 3 * D), b_hh2.reshape(1, 3 * D)),
    )
    for conv in (0, 1):
        lwT, lb, wihT, whhT, bih, bhh = params[conv]
        for step in range(STEPS):
            hw, gh = _tca(h, lwT, lb, whhT, bhh)
            a = _get_sc_segsum()(hw, srcs, dsts, bounds, zrows)
            step_fn = _tcb_relu if (conv == 0 and step == STEPS - 1) else _tcb
            h = step_fn(a, gh, h, wihT, bih)
    return _mean(h)
